# initial kernel scaffold (unmeasured)
import jax
import jax.numpy as jnp
from jax import lax
from jax.experimental import pallas as pl
from jax.experimental.pallas import tpu as pltpu

T = 1024
D = 1024
V_LOCAL = 8192
W = 8

_sem_signal = getattr(pl, "semaphore_signal", None) or pltpu.semaphore_signal
_sem_wait = getattr(pl, "semaphore_wait", None) or pltpu.semaphore_wait
_CompilerParams = getattr(pltpu, "CompilerParams", None) or pltpu.TPUCompilerParams


def kernel(ids, E):
    my_x = lax.axis_index("x")
    off = my_x * V_LOCAL
    loc = jnp.clip(ids - off, 0, V_LOCAL - 1).astype(jnp.int32)
    mask = ((ids >= off) & (ids < off + V_LOCAL)).astype(jnp.float32)[:, None]

    def body(loc_ref, mask_ref, e_ref, out_ref, send_buf, recv_buf,
             gsem, send_sem, recv_sem):
        mx = lax.axis_index("x")
        my = lax.axis_index("y")
        mz = lax.axis_index("z")
        partner = (1 - mx, my, mz)

        barrier_sem = pltpu.get_barrier_semaphore()
        _sem_signal(barrier_sem, inc=1, device_id=partner,
                    device_id_type=pl.DeviceIdType.MESH)
        _sem_wait(barrier_sem, 1)

        def gather_step(i, carry):
            slot = lax.rem(i, W)

            @pl.when(i >= W)
            def _():
                pltpu.make_async_copy(
                    e_ref.at[pl.ds(0, 1)], out_ref.at[pl.ds(0, 1)],
                    gsem.at[slot],
                ).wait()

            row = loc_ref[i]
            pltpu.make_async_copy(
                e_ref.at[pl.ds(row, 1)], out_ref.at[pl.ds(i, 1)],
                gsem.at[slot],
            ).start()
            return carry

        lax.fori_loop(0, T, gather_step, 0)
        for s in range(W):
            pltpu.make_async_copy(
                e_ref.at[pl.ds(0, 1)], out_ref.at[pl.ds(0, 1)], gsem.at[s]
            ).wait()

        out_ref[...] = out_ref[...] * mask_ref[...]
        send_buf[...] = out_ref[...].astype(jnp.bfloat16)

        rdma = pltpu.make_async_remote_copy(
            src_ref=send_buf,
            dst_ref=recv_buf,
            send_sem=send_sem,
            recv_sem=recv_sem,
            device_id=partner,
            device_id_type=pl.DeviceIdType.MESH,
        )
        rdma.start()
        rdma.wait()

        out_ref[...] = out_ref[...] + recv_buf[...].astype(jnp.float32)

    return pl.pallas_call(
        body,
        out_shape=jax.ShapeDtypeStruct((T, D), jnp.float32),
        in_specs=[
            pl.BlockSpec(memory_space=pltpu.SMEM),
            pl.BlockSpec(memory_space=pltpu.VMEM),
            pl.BlockSpec(memory_space=pltpu.ANY),
        ],
        out_specs=pl.BlockSpec(memory_space=pltpu.VMEM),
        scratch_shapes=[
            pltpu.VMEM((T, D), jnp.bfloat16),
            pltpu.VMEM((T, D), jnp.bfloat16),
            pltpu.SemaphoreType.DMA((W,)),
            pltpu.SemaphoreType.DMA,
            pltpu.SemaphoreType.DMA,
        ],
        compiler_params=_CompilerParams(collective_id=0),
    )(loc, mask, E)


# baseline (device time: 129217 ns/iter reference)
import jax
import jax.numpy as jnp
from jax import lax
from jax.experimental import pallas as pl
from jax.experimental.pallas import tpu as pltpu

T = 1024
D = 1024
V_LOCAL = 8192
W = 8

_sem_signal = getattr(pl, "semaphore_signal", None) or pltpu.semaphore_signal
_sem_wait = getattr(pl, "semaphore_wait", None) or pltpu.semaphore_wait
_CompilerParams = getattr(pltpu, "CompilerParams", None) or pltpu.TPUCompilerParams


def kernel(ids, E):
    my_x = lax.axis_index("x")
    off = my_x * V_LOCAL
    loc = jnp.clip(ids - off, 0, V_LOCAL - 1).astype(jnp.int32)
    mask = ((ids >= off) & (ids < off + V_LOCAL)).astype(jnp.float32)[:, None]

    def body(loc_ref, mask_ref, e_ref, out_ref, send_buf, recv_buf,
             gsem, send_sem, recv_sem):
        mx = lax.axis_index("x")
        my = lax.axis_index("y")
        mz = lax.axis_index("z")
        partner = (1 - mx, my, mz)

        barrier_sem = pltpu.get_barrier_semaphore()
        _sem_signal(barrier_sem, inc=1, device_id=partner,
                    device_id_type=pl.DeviceIdType.MESH)
        _sem_wait(barrier_sem, 1)

        def gather_step(i, carry):
            slot = lax.rem(i, W)

            @pl.when(i >= W)
            def _():
                pltpu.make_async_copy(
                    e_ref.at[pl.ds(0, 1)], out_ref.at[pl.ds(0, 1)],
                    gsem.at[slot],
                ).wait()

            row = loc_ref[i]
            pltpu.make_async_copy(
                e_ref.at[pl.ds(row, 1)], out_ref.at[pl.ds(i, 1)],
                gsem.at[slot],
            ).start()
            return carry

        lax.fori_loop(0, T, gather_step, 0)
        for s in range(W):
            pltpu.make_async_copy(
                e_ref.at[pl.ds(0, 1)], out_ref.at[pl.ds(0, 1)], gsem.at[s]
            ).wait()

        out_ref[...] = out_ref[...] * mask_ref[...]
        send_buf[...] = out_ref[...].astype(jnp.bfloat16)

        rdma = pltpu.make_async_remote_copy(
            src_ref=send_buf,
            dst_ref=recv_buf,
            send_sem=send_sem,
            recv_sem=recv_sem,
            device_id=partner,
            device_id_type=pl.DeviceIdType.MESH,
        )
        rdma.start()
        rdma.wait()

        out_ref[...] = out_ref[...] + recv_buf[...].astype(jnp.float32)

    return pl.pallas_call(
        body,
        out_shape=jax.ShapeDtypeStruct((T, D), jnp.float32),
        in_specs=[
            pl.BlockSpec(memory_space=pltpu.SMEM),
            pl.BlockSpec(memory_space=pltpu.VMEM),
            pl.BlockSpec(memory_space=pl.ANY),
        ],
        out_specs=pl.BlockSpec(memory_space=pltpu.VMEM),
        scratch_shapes=[
            pltpu.VMEM((T, D), jnp.bfloat16),
            pltpu.VMEM((T, D), jnp.bfloat16),
            pltpu.SemaphoreType.DMA((W,)),
            pltpu.SemaphoreType.DMA,
            pltpu.SemaphoreType.DMA,
        ],
        compiler_params=_CompilerParams(collective_id=0),
    )(loc, mask, E)


# device time: 50011 ns/iter; 2.5838x vs baseline; 2.5838x over previous
import jax
import jax.numpy as jnp
from jax import lax
from jax.experimental import pallas as pl
from jax.experimental.pallas import tpu as pltpu

T = 1024
D = 1024
V_LOCAL = 8192
CH = 8
R = T // CH

_sem_signal = getattr(pl, "semaphore_signal", None) or pltpu.semaphore_signal
_sem_wait = getattr(pl, "semaphore_wait", None) or pltpu.semaphore_wait
_CompilerParams = getattr(pltpu, "CompilerParams", None) or pltpu.TPUCompilerParams


def kernel(ids, E):
    my_x = lax.axis_index("x")
    off = my_x * V_LOCAL
    loc = jnp.clip(ids - off, 0, V_LOCAL - 1).astype(jnp.int32)
    mask = ((ids >= off) & (ids < off + V_LOCAL)).astype(jnp.float32)[:, None]

    def body(loc_ref, mask_ref, e_ref, out_ref, send_buf, recv_buf,
             gsem, send_sems, recv_sems):
        mx = lax.axis_index("x")
        my = lax.axis_index("y")
        mz = lax.axis_index("z")
        partner = (1 - mx, my, mz)

        barrier_sem = pltpu.get_barrier_semaphore()
        _sem_signal(barrier_sem, inc=1, device_id=partner,
                    device_id_type=pl.DeviceIdType.MESH)
        _sem_wait(barrier_sem, 1)

        def issue(i, carry):
            row = loc_ref[i]
            c = lax.div(i, R)
            pltpu.make_async_copy(
                e_ref.at[pl.ds(row, 1)], out_ref.at[pl.ds(i, 1)],
                gsem.at[c],
            ).start()
            return carry

        lax.fori_loop(0, T, issue, 0, unroll=8)

        rdmas = []
        for c in range(CH):
            sl = pl.ds(c * R, R)
            pltpu.make_async_copy(
                e_ref.at[pl.ds(0, R)], out_ref.at[sl], gsem.at[c]
            ).wait()
            m = out_ref[sl, :] * mask_ref[sl, :]
            out_ref[sl, :] = m
            send_buf[sl, :] = m.astype(jnp.bfloat16)
            rdma = pltpu.make_async_remote_copy(
                src_ref=send_buf.at[sl],
                dst_ref=recv_buf.at[sl],
                send_sem=send_sems.at[c],
                recv_sem=recv_sems.at[c],
                device_id=partner,
                device_id_type=pl.DeviceIdType.MESH,
            )
            rdma.start()
            rdmas.append(rdma)

        for c in range(CH):
            rdmas[c].wait_recv()
            sl = pl.ds(c * R, R)
            out_ref[sl, :] = out_ref[sl, :] + recv_buf[sl, :].astype(jnp.float32)

        for c in range(CH):
            rdmas[c].wait_send()

    return pl.pallas_call(
        body,
        out_shape=jax.ShapeDtypeStruct((T, D), jnp.float32),
        in_specs=[
            pl.BlockSpec(memory_space=pltpu.SMEM),
            pl.BlockSpec(memory_space=pltpu.VMEM),
            pl.BlockSpec(memory_space=pl.ANY),
        ],
        out_specs=pl.BlockSpec(memory_space=pltpu.VMEM),
        scratch_shapes=[
            pltpu.VMEM((T, D), jnp.bfloat16),
            pltpu.VMEM((T, D), jnp.bfloat16),
            pltpu.SemaphoreType.DMA((CH,)),
            pltpu.SemaphoreType.DMA((CH,)),
            pltpu.SemaphoreType.DMA((CH,)),
        ],
        compiler_params=_CompilerParams(collective_id=0),
    )(loc, mask, E)


# device time: 50010 ns/iter; 2.5838x vs baseline; 1.0000x over previous
import jax
import jax.numpy as jnp
from jax import lax
from jax.experimental import pallas as pl
from jax.experimental.pallas import tpu as pltpu

T = 1024
D = 1024
V_LOCAL = 8192
CH = 8
R = T // CH

_sem_signal = getattr(pl, "semaphore_signal", None) or pltpu.semaphore_signal
_sem_wait = getattr(pl, "semaphore_wait", None) or pltpu.semaphore_wait
_CompilerParams = getattr(pltpu, "CompilerParams", None) or pltpu.TPUCompilerParams


def kernel(ids, E):
    my_x = lax.axis_index("x")
    off = my_x * V_LOCAL
    loc = jnp.clip(ids - off, 0, V_LOCAL - 1).astype(jnp.int32)
    mask = ((ids >= off) & (ids < off + V_LOCAL)).astype(jnp.float32)[:, None]

    def body(loc_ref, mask_ref, e_ref, out_ref, send_buf, recv_buf,
             gsem, send_sems, recv_sems):
        mx = lax.axis_index("x")
        my = lax.axis_index("y")
        mz = lax.axis_index("z")
        partner = (1 - mx, my, mz)

        barrier_sem = pltpu.get_barrier_semaphore()
        _sem_signal(barrier_sem, inc=1, device_id=partner,
                    device_id_type=pl.DeviceIdType.MESH)
        _sem_wait(barrier_sem, 1)

        def issue_chunk(c):
            def issue(i, carry):
                row = loc_ref[i]
                pltpu.make_async_copy(
                    e_ref.at[pl.ds(row, 1)], out_ref.at[pl.ds(i, 1)],
                    gsem.at[c],
                ).start()
                return carry

            lax.fori_loop(c * R, (c + 1) * R, issue, 0, unroll=8)

        issue_chunk(0)
        issue_chunk(1)

        rdmas = []
        for c in range(CH):
            sl = pl.ds(c * R, R)
            pltpu.make_async_copy(
                e_ref.at[pl.ds(0, R)], out_ref.at[sl], gsem.at[c]
            ).wait()
            m = out_ref[sl, :] * mask_ref[sl, :]
            out_ref[sl, :] = m
            send_buf[sl, :] = m.astype(jnp.bfloat16)
            rdma = pltpu.make_async_remote_copy(
                src_ref=send_buf.at[sl],
                dst_ref=recv_buf.at[sl],
                send_sem=send_sems.at[c],
                recv_sem=recv_sems.at[c],
                device_id=partner,
                device_id_type=pl.DeviceIdType.MESH,
            )
            rdma.start()
            rdmas.append(rdma)
            if c + 2 < CH:
                issue_chunk(c + 2)

        for c in range(CH):
            rdmas[c].wait_recv()
            sl = pl.ds(c * R, R)
            out_ref[sl, :] = out_ref[sl, :] + recv_buf[sl, :].astype(jnp.float32)

        for c in range(CH):
            rdmas[c].wait_send()

    return pl.pallas_call(
        body,
        out_shape=jax.ShapeDtypeStruct((T, D), jnp.float32),
        in_specs=[
            pl.BlockSpec(memory_space=pltpu.SMEM),
            pl.BlockSpec(memory_space=pltpu.VMEM),
            pl.BlockSpec(memory_space=pl.ANY),
        ],
        out_specs=pl.BlockSpec(memory_space=pltpu.VMEM),
        scratch_shapes=[
            pltpu.VMEM((T, D), jnp.bfloat16),
            pltpu.VMEM((T, D), jnp.bfloat16),
            pltpu.SemaphoreType.DMA((CH,)),
            pltpu.SemaphoreType.DMA((CH,)),
            pltpu.SemaphoreType.DMA((CH,)),
        ],
        compiler_params=_CompilerParams(collective_id=0),
    )(loc, mask, E)
